# Initial kernel scaffold; baseline (speedup 1.0000x reference)
import jax
import jax.numpy as jnp
from jax.experimental import pallas as pl


def kernel(z_e, emb_weight, kappa_phi):
    raise NotImplementedError("write your pallas kernel here")


# speedup vs baseline: 1.0540x; 1.0540x over previous
import numpy as np
import jax
import jax.numpy as jnp
from jax import lax
from jax.experimental import pallas as pl
from jax.experimental.pallas import tpu as pltpu

_B, _D, _H, _W = 32, 64, 32, 32
_K = 1024
_N = _B * _H * _W
_R = 256
_GRID = _N // _R

_ROT = ((13, 15, 26, 6), (17, 29, 16, 24))
_KS = (np.uint32(0), np.uint32(42), np.uint32(42 ^ 0x1BD11BDA))
_TINY = np.float32(np.finfo(np.float32).tiny)


def _rotl(x, r):
    return lax.shift_left(x, np.uint32(r)) | lax.shift_right_logical(
        x, np.uint32(32 - r))


def _threefry_bits(j):
    x0 = jnp.zeros_like(j) + _KS[0]
    x1 = j + _KS[1]
    for i in range(5):
        for r in _ROT[i % 2]:
            x0 = x0 + x1
            x1 = _rotl(x1, r)
            x1 = x1 ^ x0
        x0 = x0 + _KS[(i + 1) % 3]
        x1 = x1 + _KS[(i + 2) % 3] + np.uint32(i + 1)
    return x0 ^ x1


def _body(z_ref, emb_ref, kappa_ref, zq_ref, idx_ref, reg_ref):
    t = pl.program_id(0)

    emb = emb_ref[...]
    norm = jnp.sqrt(jnp.sum(emb * emb, axis=1, keepdims=True))
    emb_n = emb / jnp.maximum(norm, np.float32(1e-12))

    zd = z_ref[0]
    sims = lax.dot_general(
        zd, emb_n, (((0,), (1,)), ((), ())),
        preferred_element_type=jnp.float32)

    kappa = kappa_ref[0, 0]
    logits = kappa * sims

    row = lax.broadcasted_iota(jnp.uint32, (_R, _K), 0)
    col = lax.broadcasted_iota(jnp.uint32, (_R, _K), 1)
    base = lax.convert_element_type(t * (_R * _K), jnp.uint32)
    j = base + row * np.uint32(_K) + col
    bits = _threefry_bits(j)

    uf = lax.bitcast_convert_type(
        lax.shift_right_logical(bits, np.uint32(9)) | np.uint32(0x3F800000),
        jnp.float32) - np.float32(1.0)
    u = jnp.maximum(_TINY, uf + _TINY)
    g = -jnp.log(-jnp.log(u))
    score = g + logits

    idx = jnp.argmax(score, axis=1)
    idx_ref[...] = idx.reshape(1, 1, _R)

    eq = col == idx.astype(jnp.uint32)[:, None]
    onehot = jnp.where(eq, np.float32(1.0), np.float32(0.0))
    chosen = jnp.sum(jnp.where(eq, sims, np.float32(0.0)), axis=1)
    part = jnp.sum(kappa * (np.float32(1.0) - chosen))

    @pl.when(t == 0)
    def _():
        reg_ref[0, 0] = np.float32(0.0)

    reg_ref[0, 0] += part

    zq_ref[...] = lax.dot_general(
        onehot, emb_n, (((1,), (0,)), ((), ())),
        preferred_element_type=jnp.float32)


def _quantize(z3, emb_weight, kappa2):
    return pl.pallas_call(
        _body,
        grid=(_GRID,),
        in_specs=[
            pl.BlockSpec((1, _D, _R), lambda t: (t // 4, 0, t % 4)),
            pl.BlockSpec((_K, _D), lambda t: (0, 0)),
            pl.BlockSpec((1, 1), lambda t: (0, 0), memory_space=pltpu.SMEM),
        ],
        out_specs=[
            pl.BlockSpec((_R, _D), lambda t: (t, 0)),
            pl.BlockSpec((1, 1, _R), lambda t: (t, 0, 0)),
            pl.BlockSpec((1, 1), lambda t: (0, 0), memory_space=pltpu.SMEM),
        ],
        out_shape=[
            jax.ShapeDtypeStruct((_N, _D), jnp.float32),
            jax.ShapeDtypeStruct((_GRID, 1, _R), jnp.int32),
            jax.ShapeDtypeStruct((1, 1), jnp.float32),
        ],
    )(z3, emb_weight, kappa2)


def kernel(z_e, emb_weight, kappa_phi):
    B, D, H, W = z_e.shape
    z3 = z_e.reshape(B, D, H * W)
    kappa2 = jnp.reshape(kappa_phi, (1, 1)).astype(jnp.float32)
    zq_flat, idx, reg_sum = _quantize(z3, emb_weight, kappa2)
    z_q = jnp.transpose(zq_flat.reshape(B, H, W, D), (0, 3, 1, 2))
    reg = (reg_sum[0, 0] / np.float32(_N)).astype(jnp.float32)
    indices = idx.reshape(B, H, W)
    return (z_q, reg, indices)
